# baseline (speedup 1.0000x reference)
import functools

import jax
import jax.numpy as jnp
from jax import lax
from jax.experimental import pallas as pl
from jax.experimental.pallas import tpu as pltpu
from jax.experimental.pallas import tpu_sc as plsc

D = 64
V = 1000000
NSAMP = 16384
TOK = 50
NC, NS = 2, 16
NW = NC * NS
SPW = NSAMP // NW
SCHUNK = 8
NCHUNKS = SPW // SCHUNK
NBUF = 2


def _make_gather():
    mesh = plsc.VectorSubcoreMesh(core_axis_name="c", subcore_axis_name="s")

    @functools.partial(
        pl.kernel,
        mesh=mesh,
        out_type=jax.ShapeDtypeStruct((NSAMP, TOK, D), jnp.float32),
        scratch_types=[
            [pltpu.VMEM((SCHUNK, TOK), jnp.int32) for _ in range(NBUF)],
            [pltpu.VMEM((SCHUNK, TOK, D), jnp.float32) for _ in range(NBUF)],
            [pltpu.SemaphoreType.DMA for _ in range(NBUF)],
            [pltpu.SemaphoreType.DMA for _ in range(NBUF)],
            [pltpu.SemaphoreType.DMA for _ in range(NBUF)],
        ],
        compiler_params=pltpu.CompilerParams(
            use_tc_tiling_on_sc=False,
            disable_bounds_checks=True,
            disable_semaphore_checks=True,
        ),
    )
    def gather_kernel(x_hbm, table_hbm, out_hbm, idx_v, rows_v, isem, gsem, osem):
        wid = lax.axis_index("s") * NC + lax.axis_index("c")

        def body(g, carry):
            icopies = []
            for b in range(NBUF):
                s0 = wid * SPW + (g * NBUF + b) * SCHUNK
                icopies.append(
                    pltpu.async_copy(x_hbm.at[pl.ds(s0, SCHUNK)], idx_v[b], isem[b])
                )
            gcopies = []
            for b in range(NBUF):
                icopies[b].wait()
                gcopies.append([
                    pltpu.async_copy(
                        table_hbm.at[idx_v[b].at[s]],
                        rows_v[b].at[s],
                        gsem[b],
                    )
                    for s in range(SCHUNK)
                ])
            ocopies = []
            for b in range(NBUF):
                s0 = wid * SPW + (g * NBUF + b) * SCHUNK
                for cp in gcopies[b]:
                    cp.wait()
                ocopies.append(
                    pltpu.async_copy(rows_v[b], out_hbm.at[pl.ds(s0, SCHUNK)], osem[b])
                )
            for cp in ocopies:
                cp.wait()
            return carry

        lax.fori_loop(0, NCHUNKS // NBUF, body, 0)

    return gather_kernel


_gather = _make_gather()


def kernel(x, embedding):
    return _gather(x.astype(jnp.int32), embedding)


# speedup vs baseline: 1.0101x; 1.0101x over previous
import functools

import jax
import jax.numpy as jnp
from jax import lax
from jax.experimental import pallas as pl
from jax.experimental.pallas import tpu as pltpu
from jax.experimental.pallas import tpu_sc as plsc

D = 64
V = 1000000
NSAMP = 16384
TOK = 50
NLOOK = NSAMP * TOK
NC, NS = 2, 16
NW = NC * NS
LPW = NLOOK // NW
C = 128
IBLK = 2560
NIB = LPW // IBLK
NBUF = 4
GRP = IBLK // C // NBUF


def _make_gather():
    mesh = plsc.VectorSubcoreMesh(core_axis_name="c", subcore_axis_name="s")

    @functools.partial(
        pl.kernel,
        mesh=mesh,
        out_type=jax.ShapeDtypeStruct((NLOOK, D), jnp.float32),
        scratch_types=[
            [pltpu.VMEM((IBLK,), jnp.int32) for _ in range(2)],
            [pltpu.VMEM((C, D), jnp.float32) for _ in range(NBUF)],
            [pltpu.SemaphoreType.DMA for _ in range(2)],
            [pltpu.SemaphoreType.DMA for _ in range(NBUF)],
            [pltpu.SemaphoreType.DMA for _ in range(NBUF)],
        ],
        compiler_params=pltpu.CompilerParams(
            use_tc_tiling_on_sc=False,
            disable_bounds_checks=True,
            disable_semaphore_checks=True,
        ),
    )
    def gather_kernel(x_hbm, table_hbm, out_hbm, idx_v, rows_v, isem, gsem, wsem):
        wid = lax.axis_index("s") * NC + lax.axis_index("c")
        base = wid * LPW

        icopies = [None, None]
        icopies[0] = pltpu.async_copy(
            x_hbm.at[pl.ds(base, IBLK)], idx_v[0], isem[0]
        )
        for ib in range(NIB):
            pb = ib % 2
            if ib + 1 < NIB:
                icopies[(ib + 1) % 2] = pltpu.async_copy(
                    x_hbm.at[pl.ds(base + (ib + 1) * IBLK, IBLK)],
                    idx_v[(ib + 1) % 2],
                    isem[(ib + 1) % 2],
                )
            icopies[pb].wait()
            bbase = base + ib * IBLK

            def body(g, carry, pb=pb, bbase=bbase):
                gcopies = []
                for b in range(NBUF):
                    off = (g * NBUF + b) * C
                    gcopies.append(
                        pltpu.async_copy(
                            table_hbm.at[idx_v[pb].at[pl.ds(off, C)]],
                            rows_v[b],
                            gsem[b],
                        )
                    )
                ocopies = []
                for b in range(NBUF):
                    off = (g * NBUF + b) * C
                    gcopies[b].wait()
                    ocopies.append(
                        pltpu.async_copy(
                            rows_v[b],
                            out_hbm.at[pl.ds(bbase + off, C)],
                            wsem[b],
                        )
                    )
                for cp in ocopies:
                    cp.wait()
                return carry

            lax.fori_loop(0, GRP, body, 0)

    return gather_kernel


_gather = _make_gather()


def kernel(x, embedding):
    out = _gather(x.reshape(NLOOK).astype(jnp.int32), embedding)
    return out.reshape(NSAMP, TOK, D)
